# baseline (speedup 1.0000x reference)
import jax
import jax.numpy as jnp
from jax.experimental import pallas as pl

_N_USER = 10000
_N_ITEM = 10000
_D = 256
_H = 256
_E = 160000
_EL = 20000


def _h_score_body(x_ref, w_ref, a_ref, h_ref, s_ref):
    h = jnp.dot(x_ref[...], w_ref[...], preferred_element_type=jnp.float32)
    h_ref[...] = h
    s_ref[...] = jnp.dot(h, a_ref[...], preferred_element_type=jnp.float32)


def _linear_and_score(x, W, a):
    N, d = x.shape
    B = 2000
    h, s = pl.pallas_call(
        _h_score_body,
        grid=(N // B,),
        in_specs=[
            pl.BlockSpec((B, d), lambda i: (i, 0)),
            pl.BlockSpec((d, _H), lambda i: (0, 0)),
            pl.BlockSpec((_H, 1), lambda i: (0, 0)),
        ],
        out_specs=[
            pl.BlockSpec((B, _H), lambda i: (i, 0)),
            pl.BlockSpec((B, 1), lambda i: (i, 0)),
        ],
        out_shape=[
            jax.ShapeDtypeStruct((N, _H), jnp.float32),
            jax.ShapeDtypeStruct((N, 1), jnp.float32),
        ],
    )(x, W, a.reshape(_H, 1))
    return h, s[:, 0]


def _dst_score_body(x_ref, w_ref, a_ref, s_ref):
    wa = jnp.dot(w_ref[...], a_ref[...], preferred_element_type=jnp.float32)
    s_ref[...] = jnp.dot(x_ref[...], wa, preferred_element_type=jnp.float32)


def _dst_score(x, W, a):
    N, d = x.shape
    B = 2000
    s = pl.pallas_call(
        _dst_score_body,
        grid=(N // B,),
        in_specs=[
            pl.BlockSpec((B, d), lambda i: (i, 0)),
            pl.BlockSpec((d, _H), lambda i: (0, 0)),
            pl.BlockSpec((_H, 1), lambda i: (0, 0)),
        ],
        out_specs=pl.BlockSpec((B, 1), lambda i: (i, 0)),
        out_shape=jax.ShapeDtypeStruct((N, 1), jnp.float32),
    )(x, W, a.reshape(_H, 1))
    return s[:, 0]


def _edge_e_body(ss_ref, sd_ref, e_ref):
    al = ss_ref[...] + sd_ref[...]
    e_ref[...] = jnp.where(al > 0, al, 0.2 * al)


def _edge_e(ss_g, sd_g):
    E = ss_g.shape[0]
    e = pl.pallas_call(
        _edge_e_body,
        in_specs=[
            pl.BlockSpec((E // 128, 128), lambda: (0, 0)),
            pl.BlockSpec((E // 128, 128), lambda: (0, 0)),
        ],
        out_specs=pl.BlockSpec((E // 128, 128), lambda: (0, 0)),
        out_shape=jax.ShapeDtypeStruct((E // 128, 128), jnp.float32),
    )(ss_g.reshape(E // 128, 128), sd_g.reshape(E // 128, 128))
    return e.reshape(E)


def _edge_ex_body(e_ref, mg_ref, ex_ref):
    ex_ref[...] = jnp.exp(e_ref[...] - mg_ref[...])


def _edge_ex(e, m_g):
    E = e.shape[0]
    ex = pl.pallas_call(
        _edge_ex_body,
        in_specs=[
            pl.BlockSpec((E // 128, 128), lambda: (0, 0)),
            pl.BlockSpec((E // 128, 128), lambda: (0, 0)),
        ],
        out_specs=pl.BlockSpec((E // 128, 128), lambda: (0, 0)),
        out_shape=jax.ShapeDtypeStruct((E // 128, 128), jnp.float32),
    )(e.reshape(E // 128, 128), m_g.reshape(E // 128, 128))
    return ex.reshape(E)


def _msg_body(ex_ref, dg_ref, h_ref, o_ref):
    alpha = ex_ref[...] / (dg_ref[...] + 1e-16)
    o_ref[...] = alpha * h_ref[...]


def _msg(ex, den_g, hs_g):
    E = ex.shape[0]
    B = 2000
    return pl.pallas_call(
        _msg_body,
        grid=(E // B,),
        in_specs=[
            pl.BlockSpec((B, 1), lambda i: (i, 0)),
            pl.BlockSpec((B, 1), lambda i: (i, 0)),
            pl.BlockSpec((B, _H), lambda i: (i, 0)),
        ],
        out_specs=pl.BlockSpec((B, _H), lambda i: (i, 0)),
        out_shape=jax.ShapeDtypeStruct((E, _H), jnp.float32),
    )(ex.reshape(E, 1), den_g.reshape(E, 1), hs_g)


def _bias_relu_body(x_ref, b_ref, o_ref):
    o_ref[...] = jnp.maximum(x_ref[...] + b_ref[...], 0.0)


def _bias_body(x_ref, b_ref, o_ref):
    o_ref[...] = x_ref[...] + b_ref[...]


def _bias(x, b, relu):
    N = x.shape[0]
    B = 2000
    return pl.pallas_call(
        _bias_relu_body if relu else _bias_body,
        grid=(N // B,),
        in_specs=[
            pl.BlockSpec((B, _H), lambda i: (i, 0)),
            pl.BlockSpec((1, _H), lambda i: (0, 0)),
        ],
        out_specs=pl.BlockSpec((B, _H), lambda i: (i, 0)),
        out_shape=jax.ShapeDtypeStruct((N, _H), jnp.float32),
    )(x, b.reshape(1, _H))


def _gat(x_src, x_dst, ei, Ws, Wd, a_s, a_d, b, n_dst, relu):
    hs, ss = _linear_and_score(x_src, Ws, a_s)
    sd = _dst_score(x_dst, Wd, a_d)
    src, col = ei[0], ei[1]
    e = _edge_e(ss[src], sd[col])
    m = jax.ops.segment_max(e, col, num_segments=n_dst)
    m = jnp.where(jnp.isfinite(m), m, 0.0)
    ex = _edge_ex(e, m[col])
    den = jax.ops.segment_sum(ex, col, num_segments=n_dst)
    msg = _msg(ex, den[col], hs[src])
    agg = jax.ops.segment_sum(msg, col, num_segments=n_dst)
    return _bias(agg, b, relu)


def _dec_body(zr_ref, zc_ref, w1a_ref, w1b_ref, b1_ref, w2_ref, o_ref):
    h = (
        jnp.dot(zr_ref[...], w1a_ref[...], preferred_element_type=jnp.float32)
        + jnp.dot(zc_ref[...], w1b_ref[...], preferred_element_type=jnp.float32)
        + b1_ref[...]
    )
    h = jnp.maximum(h, 0.0)
    o_ref[...] = jnp.dot(h, w2_ref[...], preferred_element_type=jnp.float32)


def _decode(zr, zc, Wd1, bd1, Wd2, bd2):
    EL = zr.shape[0]
    B = 2000
    out = pl.pallas_call(
        _dec_body,
        grid=(EL // B,),
        in_specs=[
            pl.BlockSpec((B, _H), lambda i: (i, 0)),
            pl.BlockSpec((B, _H), lambda i: (i, 0)),
            pl.BlockSpec((_H, _H), lambda i: (0, 0)),
            pl.BlockSpec((_H, _H), lambda i: (0, 0)),
            pl.BlockSpec((1, _H), lambda i: (0, 0)),
            pl.BlockSpec((_H, 1), lambda i: (0, 0)),
        ],
        out_specs=pl.BlockSpec((B, 1), lambda i: (i, 0)),
        out_shape=jax.ShapeDtypeStruct((EL, 1), jnp.float32),
    )(zr, zc, Wd1[:_H], Wd1[_H:], bd1.reshape(1, _H), Wd2)
    return out[:, 0] + bd2[0]


def kernel(x_user, x_item, edge_ui, edge_iu, edge_label_index,
           W1s_ui, W1d_ui, a1s_ui, a1d_ui, b1_ui,
           W1s_iu, W1d_iu, a1s_iu, a1d_iu, b1_iu,
           W2s_ui, W2d_ui, a2s_ui, a2d_ui, b2_ui,
           W2s_iu, W2d_iu, a2s_iu, a2d_iu, b2_iu,
           Wd1, bd1, Wd2, bd2):
    z_item1 = _gat(x_user, x_item, edge_ui, W1s_ui, W1d_ui, a1s_ui, a1d_ui,
                   b1_ui, _N_ITEM, relu=True)
    z_user1 = _gat(x_item, x_user, edge_iu, W1s_iu, W1d_iu, a1s_iu, a1d_iu,
                   b1_iu, _N_USER, relu=True)
    z_item = _gat(z_user1, z_item1, edge_ui, W2s_ui, W2d_ui, a2s_ui, a2d_ui,
                  b2_ui, _N_ITEM, relu=False)
    z_user = _gat(z_item1, z_user1, edge_iu, W2s_iu, W2d_iu, a2s_iu, a2d_iu,
                  b2_iu, _N_USER, relu=False)
    row, col = edge_label_index[0], edge_label_index[1]
    return _decode(z_user[row], z_item[col], Wd1, bd1, Wd2, bd2)
